# baseline (speedup 1.0000x reference)
import functools

import jax
import jax.numpy as jnp
from jax import lax
from jax.experimental import pallas as pl
from jax.experimental.pallas import tpu as pltpu
from jax.experimental.pallas import tpu_sc as plsc

_NUM_UNITS = 64
_MAX_REL = 128
_L = 2048


def _rel_pos_sc(e):
    e_rows = e.shape[0]
    nc, ns = 2, 16
    nw = nc * ns
    k_per_tile = e_rows // ns
    rows_per_worker = _L // nw

    mesh = plsc.VectorSubcoreMesh(
        core_axis_name="c", subcore_axis_name="s", num_cores=nc, num_subcores=ns
    )

    @functools.partial(
        pl.kernel,
        out_type=jax.ShapeDtypeStruct((_L, _L, _NUM_UNITS), jnp.float32),
        mesh=mesh,
        scratch_types=[
            pltpu.VMEM((k_per_tile, _NUM_UNITS), jnp.float32),
            pltpu.VMEM_SHARED((e_rows, _NUM_UNITS), jnp.float32),
        ],
        compiler_params=pltpu.CompilerParams(use_tc_tiling_on_sc=False),
    )
    def k(e_hbm, out_hbm, chunk_v, e_sh):
        c = lax.axis_index("c")
        s = lax.axis_index("s")

        k0 = s * k_per_tile
        pltpu.sync_copy(e_hbm.at[pl.ds(k0, k_per_tile)], chunk_v)
        pltpu.sync_copy(chunk_v, e_sh.at[pl.ds(k0, k_per_tile)])
        plsc.subcore_barrier()

        base = (c * ns + s) * rows_per_worker

        def body(t, carry):
            i = base + t
            pltpu.sync_copy(e_sh.at[pl.ds((_L - 1) - i, _L)], out_hbm.at[i])
            return carry

        lax.fori_loop(0, rows_per_worker, body, 0)

    return k(e)


def kernel(embeddings_table, length_q, length_k, relative_v):
    ark = jnp.arange(2 * _L, dtype=jnp.int32)
    idxk = jnp.clip(
        jnp.clip(ark - (_L - 1), -_MAX_REL, _MAX_REL) + _MAX_REL + relative_v,
        0,
        embeddings_table.shape[0] - 1,
    )
    e = jnp.take(embeddings_table, idxk, axis=0)
    return _rel_pos_sc(e)
